# baseline (speedup 1.0000x reference)
import jax
import jax.numpy as jnp
from jax import lax
from jax.experimental import pallas as pl
from jax.experimental.pallas import tpu as pltpu
from jax.experimental.pallas import tpu_sc as plsc

BATCH = 4
SEQ = 8192
D = 1024
CHUNK = 32
NW = 32
ROWS_PER_W = SEQ // NW
NCHUNK = ROWS_PER_W // CHUNK

SEQ_BLOCK = 2048
SC_BATCH = 3


def _sc_body(x_hbm, tab_hbm, pos_hbm, out_hbm, pos_v, tab_buf, x_buf, sem):
    info = plsc.get_sparse_core_info()
    wid = lax.axis_index("s") * info.num_cores + lax.axis_index("c")
    off = wid * ROWS_PER_W

    def chunk_body(ci, carry):
        row0 = off + ci * CHUNK
        pltpu.sync_copy(pos_hbm.at[pl.ds(row0, CHUNK)], pos_v)
        pltpu.async_copy(tab_hbm.at[pos_v], tab_buf, sem).wait()
        xrow0 = SC_BATCH * SEQ + row0
        pltpu.sync_copy(x_hbm.at[pl.ds(xrow0, CHUNK)], x_buf)

        @plsc.parallel_loop(0, CHUNK, 1, unroll=2)
        def r_body(r):
            for c in range(D // 16):
                sl = pl.ds(c * 16, 16)
                x_buf[r, sl] = x_buf[r, sl] + tab_buf[r, sl]

        pltpu.sync_copy(x_buf, out_hbm.at[pl.ds(row0, CHUNK)])
        return 0

    lax.fori_loop(0, NCHUNK, chunk_body, 0)


def _sc_part(xf, pos_table, pos32):
    mesh = plsc.VectorSubcoreMesh(core_axis_name="c", subcore_axis_name="s")
    return pl.kernel(
        _sc_body,
        out_type=jax.ShapeDtypeStruct((SEQ, D), jnp.float32),
        mesh=mesh,
        scratch_types=[
            pltpu.VMEM((CHUNK,), jnp.int32),
            pltpu.VMEM((CHUNK, D), jnp.float32),
            pltpu.VMEM((CHUNK, D), jnp.float32),
            pltpu.SemaphoreType.DMA,
        ],
    )(xf, pos_table, pos32)


def _tc_add_kernel(pos_ref, x_ref, tab_ref, out_ref):
    out_ref[...] = x_ref[...] + tab_ref[...]


def _tc_part(x, pos_table, pos32):
    ns = SEQ // SEQ_BLOCK
    grid_spec = pltpu.PrefetchScalarGridSpec(
        num_scalar_prefetch=1,
        grid=(ns, SC_BATCH),
        in_specs=[
            pl.BlockSpec((1, SEQ_BLOCK, D), lambda s, b, pos: (b, s, 0)),
            pl.BlockSpec(
                (SEQ_BLOCK, D),
                lambda s, b, pos: (pos[s * SEQ_BLOCK] // SEQ_BLOCK, 0),
            ),
        ],
        out_specs=pl.BlockSpec((1, SEQ_BLOCK, D), lambda s, b, pos: (b, s, 0)),
    )
    return pl.pallas_call(
        _tc_add_kernel,
        grid_spec=grid_spec,
        out_shape=jax.ShapeDtypeStruct((SC_BATCH, SEQ, D), jnp.float32),
        compiler_params=pltpu.CompilerParams(
            dimension_semantics=("arbitrary", "arbitrary"),
        ),
    )(pos32, x, pos_table)


def kernel(x, pos_table, positions):
    pos32 = positions.astype(jnp.int32)
    xf = x.reshape(BATCH * SEQ, D)
    sc_out = _sc_part(xf, pos_table, pos32)
    tc_out = _tc_part(x, pos_table, pos32)
    return jnp.concatenate([tc_out, sc_out[None]], axis=0)


# speedup vs baseline: 1.5737x; 1.5737x over previous
import jax
import jax.numpy as jnp
from jax import lax
from jax.experimental import pallas as pl
from jax.experimental.pallas import tpu as pltpu
from jax.experimental.pallas import tpu_sc as plsc

BATCH = 4
SEQ = 8192
D = 1024
CHUNK = 32
NW = 32
ROWS_PER_W = SEQ // NW
NCHUNK = ROWS_PER_W // CHUNK

SEQ_BLOCK = 2048


def _sc_gather_body(tab_hbm, pos_hbm, out_hbm, pos_v, buf_a, buf_b, buf_c,
                    gsem_a, gsem_b, gsem_c, ssem_a, ssem_b, ssem_c):
    info = plsc.get_sparse_core_info()
    wid = lax.axis_index("s") * info.num_cores + lax.axis_index("c")
    off = wid * ROWS_PER_W

    bufs = (buf_a, buf_b, buf_c)
    gsems = (gsem_a, gsem_b, gsem_c)
    ssems = (ssem_a, ssem_b, ssem_c)

    pltpu.sync_copy(pos_hbm.at[pl.ds(off, ROWS_PER_W)], pos_v)

    gathers = [None] * NCHUNK
    stores = [None] * NCHUNK

    def start_gather(ci):
        p = ci % 3
        gathers[ci] = pltpu.async_copy(
            tab_hbm.at[pos_v.at[pl.ds(ci * CHUNK, CHUNK)]], bufs[p], gsems[p])

    start_gather(0)
    start_gather(1)
    for ci in range(NCHUNK):
        p = ci % 3
        if ci + 2 < NCHUNK:
            if ci - 1 >= 0:
                stores[ci - 1].wait()
            start_gather(ci + 2)
        gathers[ci].wait()
        row0 = off + ci * CHUNK
        stores[ci] = pltpu.async_copy(bufs[p], out_hbm.at[pl.ds(row0, CHUNK)],
                                      ssems[p])
    stores[NCHUNK - 3].wait()
    stores[NCHUNK - 2].wait()
    stores[NCHUNK - 1].wait()


def _sc_gather(pos_table, pos32):
    mesh = plsc.VectorSubcoreMesh(core_axis_name="c", subcore_axis_name="s")
    return pl.kernel(
        _sc_gather_body,
        out_type=jax.ShapeDtypeStruct((SEQ, D), jnp.float32),
        mesh=mesh,
        scratch_types=[
            pltpu.VMEM((ROWS_PER_W,), jnp.int32),
            pltpu.VMEM((CHUNK, D), jnp.float32),
            pltpu.VMEM((CHUNK, D), jnp.float32),
            pltpu.VMEM((CHUNK, D), jnp.float32),
            pltpu.SemaphoreType.DMA,
            pltpu.SemaphoreType.DMA,
            pltpu.SemaphoreType.DMA,
            pltpu.SemaphoreType.DMA,
            pltpu.SemaphoreType.DMA,
            pltpu.SemaphoreType.DMA,
        ],
    )(pos_table, pos32)


def _tc_add_kernel(x_ref, emb_ref, out_ref):
    out_ref[...] = x_ref[...] + emb_ref[...]


def _tc_add(x, pos_emb):
    batch, max_len, d_model = x.shape
    ns = max_len // SEQ_BLOCK
    return pl.pallas_call(
        _tc_add_kernel,
        grid=(ns, batch),
        in_specs=[
            pl.BlockSpec((1, SEQ_BLOCK, d_model), lambda s, b: (b, s, 0)),
            pl.BlockSpec((SEQ_BLOCK, d_model), lambda s, b: (s, 0)),
        ],
        out_specs=pl.BlockSpec((1, SEQ_BLOCK, d_model), lambda s, b: (b, s, 0)),
        out_shape=jax.ShapeDtypeStruct(x.shape, x.dtype),
        compiler_params=pltpu.CompilerParams(
            dimension_semantics=("arbitrary", "arbitrary"),
        ),
    )(x, pos_emb)


def kernel(x, pos_table, positions):
    pos32 = positions.astype(jnp.int32)
    pos_emb = _sc_gather(pos_table, pos32)
    return _tc_add(x, pos_emb)


# speedup vs baseline: 1.7499x; 1.1119x over previous
import jax
import jax.numpy as jnp
from jax import lax
from jax.experimental import pallas as pl
from jax.experimental.pallas import tpu as pltpu
from jax.experimental.pallas import tpu_sc as plsc

BATCH = 4
SEQ = 8192
D = 1024
HALF = SEQ // 2

NW = 32
CHUNK = 32
ROWS_PER_W = HALF // NW
NCHUNK = ROWS_PER_W // CHUNK

SEQ_BLOCK = 2048
NS_HALF = HALF // SEQ_BLOCK


def _sc_gather_body(tab_hbm, pos_hbm, out_hbm, pos_v, buf_a, buf_b, buf_c,
                    gsem_a, gsem_b, gsem_c, ssem_a, ssem_b, ssem_c):
    info = plsc.get_sparse_core_info()
    wid = lax.axis_index("s") * info.num_cores + lax.axis_index("c")
    off = HALF + wid * ROWS_PER_W
    out0 = wid * ROWS_PER_W

    bufs = (buf_a, buf_b, buf_c)
    gsems = (gsem_a, gsem_b, gsem_c)
    ssems = (ssem_a, ssem_b, ssem_c)

    pltpu.sync_copy(pos_hbm.at[pl.ds(off, ROWS_PER_W)], pos_v)

    gathers = [None] * NCHUNK
    stores = [None] * NCHUNK

    def start_gather(ci):
        p = ci % 3
        gathers[ci] = pltpu.async_copy(
            tab_hbm.at[pos_v.at[pl.ds(ci * CHUNK, CHUNK)]], bufs[p], gsems[p])

    start_gather(0)
    start_gather(1)
    for ci in range(NCHUNK):
        p = ci % 3
        if ci + 2 < NCHUNK:
            if ci - 1 >= 0:
                stores[ci - 1].wait()
            start_gather(ci + 2)
        gathers[ci].wait()
        stores[ci] = pltpu.async_copy(
            bufs[p], out_hbm.at[pl.ds(out0 + ci * CHUNK, CHUNK)], ssems[p])
    for st in stores[max(0, NCHUNK - 3):]:
        st.wait()


def _sc_gather_half(pos_table, pos32):
    mesh = plsc.VectorSubcoreMesh(core_axis_name="c", subcore_axis_name="s")
    return pl.kernel(
        _sc_gather_body,
        out_type=jax.ShapeDtypeStruct((HALF, D), jnp.float32),
        mesh=mesh,
        scratch_types=[
            pltpu.VMEM((ROWS_PER_W,), jnp.int32),
            pltpu.VMEM((CHUNK, D), jnp.float32),
            pltpu.VMEM((CHUNK, D), jnp.float32),
            pltpu.VMEM((CHUNK, D), jnp.float32),
            pltpu.SemaphoreType.DMA,
            pltpu.SemaphoreType.DMA,
            pltpu.SemaphoreType.DMA,
            pltpu.SemaphoreType.DMA,
            pltpu.SemaphoreType.DMA,
            pltpu.SemaphoreType.DMA,
        ],
    )(pos_table, pos32)


def _tc_a_kernel(pos_ref, x_ref, tab_ref, out_ref):
    out_ref[...] = x_ref[...] + tab_ref[...]


def _tc_stage_a(pos32, x, pos_table):
    grid_spec = pltpu.PrefetchScalarGridSpec(
        num_scalar_prefetch=1,
        grid=(NS_HALF, BATCH),
        in_specs=[
            pl.BlockSpec((1, SEQ_BLOCK, D), lambda s, b, pos: (b, s, 0)),
            pl.BlockSpec(
                (SEQ_BLOCK, D),
                lambda s, b, pos: (pos[s * SEQ_BLOCK] // SEQ_BLOCK, 0),
            ),
        ],
        out_specs=pl.BlockSpec((1, SEQ_BLOCK, D), lambda s, b, pos: (b, s, 0)),
    )
    return pl.pallas_call(
        _tc_a_kernel,
        grid_spec=grid_spec,
        out_shape=jax.ShapeDtypeStruct((BATCH, SEQ, D), jnp.float32),
        compiler_params=pltpu.CompilerParams(
            dimension_semantics=("arbitrary", "arbitrary"),
        ),
    )(pos32, x, pos_table)


def _tc_b_kernel(x_ref, emb_ref, prev_ref, out_ref):
    out_ref[...] = x_ref[...] + emb_ref[...]


def _tc_stage_b(x, emb_b, prev):
    return pl.pallas_call(
        _tc_b_kernel,
        grid=(NS_HALF, BATCH),
        in_specs=[
            pl.BlockSpec((1, SEQ_BLOCK, D), lambda s, b: (b, s + NS_HALF, 0)),
            pl.BlockSpec((SEQ_BLOCK, D), lambda s, b: (s, 0)),
            pl.BlockSpec(memory_space=pltpu.MemorySpace.HBM),
        ],
        out_specs=pl.BlockSpec((1, SEQ_BLOCK, D), lambda s, b: (b, s + NS_HALF, 0)),
        out_shape=jax.ShapeDtypeStruct((BATCH, SEQ, D), jnp.float32),
        input_output_aliases={2: 0},
        compiler_params=pltpu.CompilerParams(
            dimension_semantics=("arbitrary", "arbitrary"),
        ),
    )(x, emb_b, prev)


def kernel(x, pos_table, positions):
    pos32 = positions.astype(jnp.int32)
    emb_b = _sc_gather_half(pos_table, pos32)
    out_a = _tc_stage_a(pos32, x, pos_table)
    return _tc_stage_b(x, emb_b, out_a)


# speedup vs baseline: 1.8391x; 1.0510x over previous
import jax
import jax.numpy as jnp
from jax import lax
from jax.experimental import pallas as pl
from jax.experimental.pallas import tpu as pltpu
from jax.experimental.pallas import tpu_sc as plsc

BATCH = 4
SEQ = 8192
D = 1024
SC_ROWS = SEQ // 4
TC_A_ROWS = SEQ - SC_ROWS

NW = 32
CHUNK = 32
ROWS_PER_W = SC_ROWS // NW
NCHUNK = ROWS_PER_W // CHUNK

SEQ_BLOCK = 2048
NS_A = TC_A_ROWS // SEQ_BLOCK
NS_B = SC_ROWS // SEQ_BLOCK


def _sc_gather_body(tab_hbm, pos_hbm, out_hbm, pos_v, buf_a, buf_b, buf_c,
                    gsem_a, gsem_b, gsem_c, ssem_a, ssem_b, ssem_c):
    info = plsc.get_sparse_core_info()
    wid = lax.axis_index("s") * info.num_cores + lax.axis_index("c")
    off = TC_A_ROWS + wid * ROWS_PER_W
    out0 = wid * ROWS_PER_W

    bufs = (buf_a, buf_b, buf_c)
    gsems = (gsem_a, gsem_b, gsem_c)
    ssems = (ssem_a, ssem_b, ssem_c)

    pltpu.sync_copy(pos_hbm.at[pl.ds(off, ROWS_PER_W)], pos_v)

    gathers = [None] * NCHUNK
    stores = [None] * NCHUNK

    def start_gather(ci):
        p = ci % 3
        gathers[ci] = pltpu.async_copy(
            tab_hbm.at[pos_v.at[pl.ds(ci * CHUNK, CHUNK)]], bufs[p], gsems[p])

    start_gather(0)
    start_gather(1)
    for ci in range(NCHUNK):
        p = ci % 3
        if ci + 2 < NCHUNK:
            if ci - 1 >= 0:
                stores[ci - 1].wait()
            start_gather(ci + 2)
        gathers[ci].wait()
        stores[ci] = pltpu.async_copy(
            bufs[p], out_hbm.at[pl.ds(out0 + ci * CHUNK, CHUNK)], ssems[p])
    for st in stores[max(0, NCHUNK - 3):]:
        st.wait()


def _sc_gather_half(pos_table, pos32):
    mesh = plsc.VectorSubcoreMesh(core_axis_name="c", subcore_axis_name="s")
    return pl.kernel(
        _sc_gather_body,
        out_type=jax.ShapeDtypeStruct((SC_ROWS, D), jnp.float32),
        mesh=mesh,
        scratch_types=[
            pltpu.VMEM((ROWS_PER_W,), jnp.int32),
            pltpu.VMEM((CHUNK, D), jnp.float32),
            pltpu.VMEM((CHUNK, D), jnp.float32),
            pltpu.VMEM((CHUNK, D), jnp.float32),
            pltpu.SemaphoreType.DMA,
            pltpu.SemaphoreType.DMA,
            pltpu.SemaphoreType.DMA,
            pltpu.SemaphoreType.DMA,
            pltpu.SemaphoreType.DMA,
            pltpu.SemaphoreType.DMA,
        ],
    )(pos_table, pos32)


def _tc_a_kernel(pos_ref, x_ref, tab_ref, out_ref):
    out_ref[...] = x_ref[...] + tab_ref[...]


def _tc_stage_a(pos32, x, pos_table):
    grid_spec = pltpu.PrefetchScalarGridSpec(
        num_scalar_prefetch=1,
        grid=(NS_A, BATCH),
        in_specs=[
            pl.BlockSpec((1, SEQ_BLOCK, D), lambda s, b, pos: (b, s, 0)),
            pl.BlockSpec(
                (SEQ_BLOCK, D),
                lambda s, b, pos: (pos[s * SEQ_BLOCK] // SEQ_BLOCK, 0),
            ),
        ],
        out_specs=pl.BlockSpec((1, SEQ_BLOCK, D), lambda s, b, pos: (b, s, 0)),
    )
    return pl.pallas_call(
        _tc_a_kernel,
        grid_spec=grid_spec,
        out_shape=jax.ShapeDtypeStruct((BATCH, SEQ, D), jnp.float32),
        compiler_params=pltpu.CompilerParams(
            dimension_semantics=("arbitrary", "arbitrary"),
        ),
    )(pos32, x, pos_table)


def _tc_b_kernel(x_ref, emb_ref, prev_ref, out_ref):
    out_ref[...] = x_ref[...] + emb_ref[...]


def _tc_stage_b(x, emb_b, prev):
    return pl.pallas_call(
        _tc_b_kernel,
        grid=(NS_B, BATCH),
        in_specs=[
            pl.BlockSpec((1, SEQ_BLOCK, D), lambda s, b: (b, s + NS_A, 0)),
            pl.BlockSpec((SEQ_BLOCK, D), lambda s, b: (s, 0)),
            pl.BlockSpec(memory_space=pltpu.MemorySpace.HBM),
        ],
        out_specs=pl.BlockSpec((1, SEQ_BLOCK, D), lambda s, b: (b, s + NS_A, 0)),
        out_shape=jax.ShapeDtypeStruct((BATCH, SEQ, D), jnp.float32),
        input_output_aliases={2: 0},
        compiler_params=pltpu.CompilerParams(
            dimension_semantics=("arbitrary", "arbitrary"),
        ),
    )(x, emb_b, prev)


def kernel(x, pos_table, positions):
    pos32 = positions.astype(jnp.int32)
    emb_b = _sc_gather_half(pos_table, pos32)
    out_a = _tc_stage_a(pos32, x, pos_table)
    return _tc_stage_b(x, emb_b, out_a)
